# Initial kernel scaffold; baseline (speedup 1.0000x reference)
import jax
import jax.numpy as jnp
from jax.experimental import pallas as pl


def kernel(x1_nchw, x2_nchw, w1, b1, w2, b2):
    raise NotImplementedError("write your pallas kernel here")


# speedup vs baseline: 1.2648x; 1.2648x over previous
import jax
import jax.numpy as jnp
from jax.experimental import pallas as pl
from jax.experimental.pallas import tpu as pltpu


def _block_kernel(x1e_ref, x2_ref, w1a_ref, w1b_ref, b1_ref, w2_ref, b2_ref,
                  o_ref):
    ws, cin = x1e_ref.shape[2], x1e_ref.shape[3]
    th = x2_ref.shape[1]
    c2 = x2_ref.shape[3]
    cout = b1_ref.shape[1]

    s = pl.program_id(1)
    nlow = th // 2 + 2

    low = x1e_ref[0, pl.ds(s * (th // 2), nlow)]
    uph = jnp.concatenate([low[:, None], low[:, None]],
                          axis=1).reshape(2 * nlow, ws, cin)

    uphm1 = jnp.concatenate([uph[:, :1], uph[:, :ws - 1]], axis=1)
    uphp1 = jnp.concatenate([uph[:, 1:], uph[:, ws - 1:]], axis=1)

    ops1a = jnp.concatenate([uphm1, uph], axis=2)
    ops1b = jnp.concatenate([uph, uphp1], axis=2)

    def conv3(ops, w_ref, rows, k):
        acc = None
        for ky in range(3):
            a = ops[ky:ky + rows].reshape(rows * ws, k)
            p = jnp.dot(a, w_ref[ky], preferred_element_type=jnp.float32)
            acc = p if acc is None else acc + p
        return acc.reshape(rows, ws, -1)

    y1a = conv3(ops1a, w1a_ref, th + 2, 2 * cin)
    y1b = conv3(ops1b, w1b_ref, th + 2, 2 * cin)

    first = s == 0
    last = s == pl.num_programs(1) - 1
    y1a = jnp.where(first, jnp.concatenate([y1a[2:3], y1a[1:]], axis=0), y1a)
    y1b = jnp.where(first, jnp.concatenate([y1b[2:3], y1b[1:]], axis=0), y1b)
    y1a = jnp.where(last, jnp.concatenate([y1a[:th + 1], y1a[th - 1:th]],
                                          axis=0), y1a)
    y1b = jnp.where(last, jnp.concatenate([y1b[:th + 1], y1b[th - 1:th]],
                                          axis=0), y1b)

    b1v = b1_ref[...]
    a1a = jnp.maximum(y1a + b1v, 0.0).astype(jnp.bfloat16)
    a1b = jnp.maximum(y1b + b1v, 0.0).astype(jnp.bfloat16)

    a1bm1 = jnp.concatenate([a1b[:, :1], a1b[:, :ws - 1]], axis=1)
    a1ap1 = jnp.concatenate([a1a[:, 1:], a1a[:, ws - 1:]], axis=1)
    ops2a = jnp.concatenate([a1bm1, a1a, a1b], axis=2)
    ops2b = jnp.concatenate([a1a, a1b, a1ap1], axis=2)

    y2a = conv3(ops2a, w2_ref, th, 3 * cout)
    y2b = conv3(ops2b, w2_ref, th, 3 * cout)

    b2v = b2_ref[...]
    y2a = jnp.maximum(y2a + b2v, 0.0)
    y2b = jnp.maximum(y2b + b2v, 0.0)

    y2 = jnp.concatenate([y2a[:, :, None], y2b[:, :, None]],
                         axis=2).reshape(th, 2 * ws, cout)
    o_ref[0, :, :, :c2] = x2_ref[0]
    o_ref[0, :, :, c2:] = y2.astype(o_ref.dtype)


def _tile_rows(hu, cap=32):
    th = min(hu, cap)
    while hu % th != 0 or th % 2 != 0:
        th -= 1
    return max(th, 2)


def kernel(x1_nchw, x2_nchw, w1, b1, w2, b2):
    n, cin, hs, ws = x1_nchw.shape
    cout = w1.shape[-1]
    c2 = x2_nchw.shape[1]
    hu, wu = 2 * hs, 2 * ws

    th = _tile_rows(hu)
    n_strips = hu // th

    x1 = jnp.transpose(x1_nchw, (0, 2, 3, 1)).astype(jnp.bfloat16)
    x1e = jnp.pad(x1, ((0, 0), (1, 1), (0, 0), (0, 0)), mode="edge")
    x2 = jnp.transpose(x2_nchw, (0, 2, 3, 1))

    w1a = jnp.concatenate([w1[:, 0], w1[:, 1] + w1[:, 2]],
                          axis=1).astype(jnp.bfloat16)
    w1b = jnp.concatenate([w1[:, 0] + w1[:, 1], w1[:, 2]],
                          axis=1).astype(jnp.bfloat16)
    w2s = w2.reshape(3, 3 * cout, cout).astype(jnp.bfloat16)
    b1r = b1.reshape(1, cout)
    b2r = b2.reshape(1, cout)

    out_nhwc = pl.pallas_call(
        _block_kernel,
        out_shape=jax.ShapeDtypeStruct((n, hu, wu, c2 + cout), x1_nchw.dtype),
        grid=(n, n_strips),
        in_specs=[
            pl.BlockSpec((1, hs + 2, ws, cin), lambda b, s: (b, 0, 0, 0)),
            pl.BlockSpec((1, th, wu, c2), lambda b, s: (b, s, 0, 0)),
            pl.BlockSpec((3, 2 * cin, cout), lambda b, s: (0, 0, 0)),
            pl.BlockSpec((3, 2 * cin, cout), lambda b, s: (0, 0, 0)),
            pl.BlockSpec((1, cout), lambda b, s: (0, 0)),
            pl.BlockSpec((3, 3 * cout, cout), lambda b, s: (0, 0, 0)),
            pl.BlockSpec((1, cout), lambda b, s: (0, 0)),
        ],
        out_specs=pl.BlockSpec((1, th, wu, c2 + cout),
                               lambda b, s: (b, s, 0, 0)),
        compiler_params=pltpu.CompilerParams(
            dimension_semantics=("parallel", "parallel"),
            vmem_limit_bytes=48 * 2 ** 20),
    )(x1e, x2, w1a, w1b, b1r, w2s, b2r)

    return jnp.transpose(out_nhwc, (0, 3, 1, 2))
